# Initial kernel scaffold; baseline (speedup 1.0000x reference)
import jax
import jax.numpy as jnp
from jax.experimental import pallas as pl


def kernel(x, edge_index, edge_type, edge_weight, basis_weights, w_comp):
    raise NotImplementedError("write your pallas kernel here")


# speedup vs baseline: 16.1268x; 16.1268x over previous
import functools

import jax
import jax.numpy as jnp
from jax import lax
from jax.experimental import pallas as pl
from jax.experimental.pallas import tpu as pltpu
from jax.experimental.pallas import tpu_sc as plsc

N_NODES = 10000
N_EDGES = 320000
IN_FEAT = 128
OUT_FEAT = 128
NUM_RELS = 8
NUM_BASES = 4

NC = 2
NS = 16
NW = NC * NS
E_PER_W = N_EDGES // NW
CHUNK = 80
N_CHUNKS = E_PER_W // CHUNK
ZROWS = 624
ZLAST = N_NODES - (NS - 1) * ZROWS
LANES = 16

ROW_BLK = 2000
N_ROW_BLKS = N_NODES // ROW_BLK


def _z_body(coef_ref, x_ref, basis_ref, z_ref):
    r = pl.program_id(0)
    rr = pl.multiple_of(r * (IN_FEAT // NUM_RELS), 8)
    w = jnp.zeros((IN_FEAT, OUT_FEAT), jnp.float32)
    for b in range(NUM_BASES):
        sb = basis_ref[b, pl.ds(rr, IN_FEAT // NUM_RELS), :]
        eb = jnp.broadcast_to(
            sb[:, None, :], (IN_FEAT // NUM_RELS, NUM_RELS, OUT_FEAT)
        ).reshape(IN_FEAT, OUT_FEAT)
        w = w + coef_ref[:, b:b + 1] * eb
    z_ref[...] = jnp.dot(x_ref[...], w, preferred_element_type=jnp.float32)


_z_call = pl.pallas_call(
    _z_body,
    grid=(NUM_RELS, N_ROW_BLKS),
    in_specs=[
        pl.BlockSpec((IN_FEAT, NUM_BASES), lambda r, i: (0, 0)),
        pl.BlockSpec((ROW_BLK, IN_FEAT), lambda r, i: (i, 0)),
        pl.BlockSpec((NUM_BASES, IN_FEAT, OUT_FEAT), lambda r, i: (0, 0, 0)),
    ],
    out_specs=pl.BlockSpec((ROW_BLK, OUT_FEAT),
                           lambda r, i: (r * N_ROW_BLKS + i, 0)),
    out_shape=jax.ShapeDtypeStruct((NUM_RELS * N_NODES, OUT_FEAT), jnp.float32),
)


_sc_mesh = plsc.VectorSubcoreMesh(core_axis_name="c", subcore_axis_name="s")


@functools.partial(
    pl.kernel,
    out_type=jax.ShapeDtypeStruct((NC, N_NODES, OUT_FEAT), jnp.float32),
    mesh=_sc_mesh,
    compiler_params=pltpu.CompilerParams(needs_layout_passes=False),
    scratch_types=[
        pltpu.VMEM((E_PER_W,), jnp.int32),
        pltpu.VMEM((E_PER_W,), jnp.int32),
        pltpu.VMEM((E_PER_W,), jnp.int32),
        pltpu.VMEM((E_PER_W,), jnp.float32),
        pltpu.VMEM((CHUNK, OUT_FEAT), jnp.float32),
        pltpu.VMEM((CHUNK,), jnp.int32),
        pltpu.VMEM_SHARED((N_NODES, OUT_FEAT), jnp.float32),
    ],
)
def _sc_scatter(z_hbm, src_hbm, dst_hbm, et_hbm, ew_hbm, zeros_hbm, p_hbm,
                zidx_v, s_v, t_v, w_v, rows_v, dst_v, accum):
    c = lax.axis_index("c")
    s = lax.axis_index("s")
    wid = s * NC + c
    ebase = wid * E_PER_W
    rbase = pl.multiple_of(s * ZROWS, 8)

    pltpu.sync_copy(src_hbm.at[pl.ds(ebase, E_PER_W)], s_v)
    pltpu.sync_copy(et_hbm.at[pl.ds(ebase, E_PER_W)], t_v)
    pltpu.sync_copy(ew_hbm.at[pl.ds(ebase, E_PER_W)], w_v)

    def idx_body(i, carry):
        sl = pl.ds(i * LANES, LANES)
        zidx_v[sl] = t_v[sl] * N_NODES + s_v[sl]
        return carry

    lax.fori_loop(0, E_PER_W // LANES, idx_body, 0)

    @pl.when(s < NS - 1)
    def _zero_main():
        pltpu.sync_copy(zeros_hbm.at[pl.ds(0, ZROWS)],
                        accum.at[pl.ds(rbase, ZROWS)])

    @pl.when(s == NS - 1)
    def _zero_last():
        pltpu.sync_copy(zeros_hbm, accum.at[pl.ds(rbase, ZLAST)])

    plsc.subcore_barrier()

    def chunk_body(ci, carry):
        off = ci * CHUNK
        pltpu.sync_copy(dst_hbm.at[pl.ds(ebase + off, CHUNK)], dst_v)
        pltpu.sync_copy(z_hbm.at[zidx_v.at[pl.ds(off, CHUNK)]], rows_v)

        def edge_body(j, carry2):
            wb = plsc.load_gather(
                w_v, [jnp.full((LANES,), off + j, jnp.int32)])
            for f in range(OUT_FEAT // LANES):
                sl = pl.ds(f * LANES, LANES)
                rows_v[j, sl] = rows_v[j, sl] * wb
            return carry2

        lax.fori_loop(0, CHUNK, edge_body, 0)
        pltpu.sync_copy(rows_v, accum.at[dst_v], add=True)
        return carry

    lax.fori_loop(0, N_CHUNKS, chunk_body, 0)
    plsc.subcore_barrier()

    @pl.when(s < NS - 1)
    def _out_main():
        pltpu.sync_copy(accum.at[pl.ds(rbase, ZROWS)],
                        p_hbm.at[c, pl.ds(rbase, ZROWS)])

    @pl.when(s == NS - 1)
    def _out_last():
        pltpu.sync_copy(accum.at[pl.ds(rbase, ZLAST)],
                        p_hbm.at[c, pl.ds(rbase, ZLAST)])


def _add_body(p_ref, o_ref):
    o_ref[...] = p_ref[0] + p_ref[1]


_add_call = pl.pallas_call(
    _add_body,
    grid=(N_ROW_BLKS,),
    in_specs=[pl.BlockSpec((NC, ROW_BLK, OUT_FEAT), lambda i: (0, i, 0))],
    out_specs=pl.BlockSpec((ROW_BLK, OUT_FEAT), lambda i: (i, 0)),
    out_shape=jax.ShapeDtypeStruct((N_NODES, OUT_FEAT), jnp.float32),
)


def kernel(x, edge_index, edge_type, edge_weight, basis_weights, w_comp):
    coef = jnp.tile(w_comp, (IN_FEAT // NUM_RELS, 1))
    z = _z_call(coef, x, basis_weights)
    zeros = jnp.zeros((ZLAST, OUT_FEAT), jnp.float32)
    p = _sc_scatter(z, edge_index[0], edge_index[1], edge_type, edge_weight,
                    zeros)
    return _add_call(p)


# speedup vs baseline: 31.1706x; 1.9328x over previous
import functools

import jax
import jax.numpy as jnp
from jax import lax
from jax.experimental import pallas as pl
from jax.experimental.pallas import tpu as pltpu
from jax.experimental.pallas import tpu_sc as plsc

N_NODES = 10000
N_EDGES = 320000
IN_FEAT = 128
OUT_FEAT = 128
NUM_RELS = 8
NUM_BASES = 4

NC = 2
NS = 16
NW = NC * NS
E_PER_W = N_EDGES // NW
CHUNK = 80
N_CHUNKS = E_PER_W // CHUNK
ZROWS = 624
ZLAST = N_NODES - (NS - 1) * ZROWS
LANES = 16

ROW_BLK = 2000
N_ROW_BLKS = N_NODES // ROW_BLK


def _z_body(coef_ref, x_ref, basis_ref, z_ref):
    r = pl.program_id(0)
    rr = pl.multiple_of(r * (IN_FEAT // NUM_RELS), 8)
    w = jnp.zeros((IN_FEAT, OUT_FEAT), jnp.float32)
    for b in range(NUM_BASES):
        sb = basis_ref[b, pl.ds(rr, IN_FEAT // NUM_RELS), :]
        eb = jnp.broadcast_to(
            sb[:, None, :], (IN_FEAT // NUM_RELS, NUM_RELS, OUT_FEAT)
        ).reshape(IN_FEAT, OUT_FEAT)
        w = w + coef_ref[:, b:b + 1] * eb
    z_ref[...] = jnp.dot(x_ref[...], w, preferred_element_type=jnp.float32)


_z_call = pl.pallas_call(
    _z_body,
    grid=(NUM_RELS, N_ROW_BLKS),
    in_specs=[
        pl.BlockSpec((IN_FEAT, NUM_BASES), lambda r, i: (0, 0)),
        pl.BlockSpec((ROW_BLK, IN_FEAT), lambda r, i: (i, 0)),
        pl.BlockSpec((NUM_BASES, IN_FEAT, OUT_FEAT), lambda r, i: (0, 0, 0)),
    ],
    out_specs=pl.BlockSpec((ROW_BLK, OUT_FEAT),
                           lambda r, i: (r * N_ROW_BLKS + i, 0)),
    out_shape=jax.ShapeDtypeStruct((NUM_RELS * N_NODES, OUT_FEAT), jnp.float32),
)


_sc_mesh = plsc.VectorSubcoreMesh(core_axis_name="c", subcore_axis_name="s")


@functools.partial(
    pl.kernel,
    out_type=jax.ShapeDtypeStruct((NC, N_NODES, OUT_FEAT), jnp.float32),
    mesh=_sc_mesh,
    compiler_params=pltpu.CompilerParams(needs_layout_passes=False),
    scratch_types=[
        pltpu.VMEM((E_PER_W,), jnp.int32),
        pltpu.VMEM((E_PER_W,), jnp.int32),
        pltpu.VMEM((E_PER_W,), jnp.float32),
        pltpu.VMEM((CHUNK, OUT_FEAT), jnp.float32),
        pltpu.VMEM((CHUNK, OUT_FEAT), jnp.float32),
        pltpu.VMEM((CHUNK,), jnp.int32),
        pltpu.VMEM((CHUNK,), jnp.int32),
        pltpu.VMEM_SHARED((N_NODES, OUT_FEAT), jnp.float32),
        pltpu.SemaphoreType.DMA,
        pltpu.SemaphoreType.DMA,
        pltpu.SemaphoreType.DMA,
        pltpu.SemaphoreType.DMA,
    ],
)
def _sc_scatter(z_hbm, src_hbm, dst_hbm, et_hbm, ew_hbm, zeros_hbm, p_hbm,
                zidx_v, t_v, w_v, rows_a, rows_b, dst_a, dst_b, accum,
                gsem_a, gsem_b, dsem_a, dsem_b):
    c = lax.axis_index("c")
    s = lax.axis_index("s")
    wid = s * NC + c
    ebase = wid * E_PER_W
    rbase = pl.multiple_of(s * ZROWS, 8)

    pltpu.sync_copy(src_hbm.at[pl.ds(ebase, E_PER_W)], zidx_v)
    pltpu.sync_copy(et_hbm.at[pl.ds(ebase, E_PER_W)], t_v)
    pltpu.sync_copy(ew_hbm.at[pl.ds(ebase, E_PER_W)], w_v)

    def idx_body(i, carry):
        sl = pl.ds(i * LANES, LANES)
        zidx_v[sl] = t_v[sl] * N_NODES + zidx_v[sl]
        return carry

    lax.fori_loop(0, E_PER_W // LANES, idx_body, 0)

    @pl.when(s < NS - 1)
    def _zero_main():
        pltpu.sync_copy(zeros_hbm.at[pl.ds(0, ZROWS)],
                        accum.at[pl.ds(rbase, ZROWS)])

    @pl.when(s == NS - 1)
    def _zero_last():
        pltpu.sync_copy(zeros_hbm, accum.at[pl.ds(rbase, ZLAST)])

    plsc.subcore_barrier()

    def start_chunk(ci, rows, dstb, gsem, dsem):
        off = ci * CHUNK
        pltpu.async_copy(dst_hbm.at[pl.ds(ebase + off, CHUNK)], dstb, dsem)
        pltpu.async_copy(z_hbm.at[zidx_v.at[pl.ds(off, CHUNK)]], rows, gsem)

    def finish_chunk(ci, rows, dstb, gsem, dsem):
        off = ci * CHUNK
        pltpu.make_async_copy(dst_hbm.at[pl.ds(ebase + off, CHUNK)],
                              dstb, dsem).wait()
        pltpu.make_async_copy(z_hbm.at[zidx_v.at[pl.ds(off, CHUNK)]],
                              rows, gsem).wait()

        def scale_body(k, carry2):
            w16 = w_v[pl.ds(off + k * LANES, LANES)]
            for j in range(LANES):
                wb = w16[jnp.full((LANES,), j, jnp.int32)]
                e = k * LANES + j
                for f in range(OUT_FEAT // LANES):
                    sl = pl.ds(f * LANES, LANES)
                    rows[e, sl] = rows[e, sl] * wb
            return carry2

        lax.fori_loop(0, CHUNK // LANES, scale_body, 0)
        pltpu.sync_copy(rows, accum.at[dstb], add=True)

    start_chunk(0, rows_a, dst_a, gsem_a, dsem_a)
    start_chunk(1, rows_b, dst_b, gsem_b, dsem_b)

    def pair_body(i, carry):
        ca = 2 * i
        finish_chunk(ca, rows_a, dst_a, gsem_a, dsem_a)

        @pl.when(ca + 2 < N_CHUNKS)
        def _():
            start_chunk(ca + 2, rows_a, dst_a, gsem_a, dsem_a)

        finish_chunk(ca + 1, rows_b, dst_b, gsem_b, dsem_b)

        @pl.when(ca + 3 < N_CHUNKS)
        def _():
            start_chunk(ca + 3, rows_b, dst_b, gsem_b, dsem_b)

        return carry

    lax.fori_loop(0, N_CHUNKS // 2, pair_body, 0)
    if N_CHUNKS % 2:
        finish_chunk(N_CHUNKS - 1, rows_a, dst_a, gsem_a, dsem_a)
    plsc.subcore_barrier()

    @pl.when(s < NS - 1)
    def _out_main():
        pltpu.sync_copy(accum.at[pl.ds(rbase, ZROWS)],
                        p_hbm.at[c, pl.ds(rbase, ZROWS)])

    @pl.when(s == NS - 1)
    def _out_last():
        pltpu.sync_copy(accum.at[pl.ds(rbase, ZLAST)],
                        p_hbm.at[c, pl.ds(rbase, ZLAST)])


def _add_body(p_ref, o_ref):
    o_ref[...] = p_ref[0] + p_ref[1]


_add_call = pl.pallas_call(
    _add_body,
    grid=(N_ROW_BLKS,),
    in_specs=[pl.BlockSpec((NC, ROW_BLK, OUT_FEAT), lambda i: (0, i, 0))],
    out_specs=pl.BlockSpec((ROW_BLK, OUT_FEAT), lambda i: (i, 0)),
    out_shape=jax.ShapeDtypeStruct((N_NODES, OUT_FEAT), jnp.float32),
)


def kernel(x, edge_index, edge_type, edge_weight, basis_weights, w_comp):
    coef = jnp.tile(w_comp, (IN_FEAT // NUM_RELS, 1))
    z = _z_call(coef, x, basis_weights)
    zeros = jnp.zeros((ZLAST, OUT_FEAT), jnp.float32)
    p = _sc_scatter(z, edge_index[0], edge_index[1], edge_type, edge_weight,
                    zeros)
    return _add_call(p)
